# baseline (speedup 1.0000x reference)
import functools

import jax
import jax.numpy as jnp
import numpy as np
from jax.experimental import pallas as pl
from jax.experimental.pallas import tpu as pltpu
from jax.experimental.pallas import tpu_sc as plsc

H = 16
HD = 64
WINDOW = 8
NEG = -1e30
TR = 256
LW = 384
GMAX = 128

_INTERPRET = False


def _matmul_kernel(x_ref, w_ref, b_ref, o_ref):
    acc = (
        jnp.dot(x_ref[...], w_ref[...], preferred_element_type=jnp.float32)
        + b_ref[...]
    )
    o_ref[...] = acc.astype(o_ref.dtype)


def _matmul(x, w, b, out_dtype=jnp.float32, bn=1024):
    m, k = x.shape
    k2, n = w.shape
    grid = (n // bn,)
    return pl.pallas_call(
        _matmul_kernel,
        grid=grid,
        in_specs=[
            pl.BlockSpec((m, k), lambda j: (0, 0)),
            pl.BlockSpec((k, bn), lambda j: (0, j)),
            pl.BlockSpec((1, bn), lambda j: (0, j)),
        ],
        out_specs=pl.BlockSpec((m, bn), lambda j: (0, j)),
        out_shape=jax.ShapeDtypeStruct((m, n), out_dtype),
        interpret=_INTERPRET,
    )(x, w, b)


def _sc_gather_rows(src, idx):
    t, c = src.shape
    g = idx.shape[0]
    b_per_w = 8
    mesh = plsc.VectorSubcoreMesh(core_axis_name="c", subcore_axis_name="s")

    @functools.partial(
        pl.kernel, mesh=mesh,
        out_type=jax.ShapeDtypeStruct((g, c), jnp.float32),
        scratch_types=[
            pltpu.VMEM((b_per_w,), jnp.int32),
            pltpu.VMEM((b_per_w, c), jnp.float32),
            pltpu.SemaphoreType.DMA,
        ],
    )
    def k(table_hbm, idx_hbm, out_hbm, idx_v, rows_v, sem):
        cid = jax.lax.axis_index("c")
        sid = jax.lax.axis_index("s")

        @pl.when(cid == 0)
        def _():
            base = sid * b_per_w
            pltpu.sync_copy(idx_hbm.at[pl.ds(base, b_per_w)], idx_v)
            pltpu.async_copy(table_hbm.at[idx_v], rows_v, sem).wait()
            pltpu.sync_copy(rows_v, out_hbm.at[pl.ds(base, b_per_w)])

    return k(src, idx)


def _gather_rows(src_bf16, idx):
    t, c = src_bf16.shape
    g = idx.shape[0]
    src_f32 = jax.lax.bitcast_convert_type(
        src_bf16.reshape(t, c // 2, 2), jnp.float32
    )
    out_f32 = _sc_gather_rows(src_f32, idx)
    return jax.lax.bitcast_convert_type(out_f32, jnp.bfloat16).reshape(g, c)


def _attn_kernel(q_ref, k_ref, v_ref, kg_ref, vg_ref, gv_ref, wo_ref, bo_ref,
                 o_ref, *, t):
    scale = 1.0 / np.sqrt(HD)
    h = pl.program_id(0)
    nr = t // TR

    q = q_ref[...].reshape(t, HD)
    k = k_ref[...].reshape(t, HD)
    v = v_ref[...].reshape(t, HD)
    kg = kg_ref[...].reshape(kg_ref.shape[0], HD)
    vg = vg_ref[...].reshape(vg_ref.shape[0], HD)
    gv = gv_ref[...] > 0.0

    ctx_rows = []
    for r in range(nr):
        t0 = r * TR
        ls = min(max(t0 - (LW - TR) // 2, 0), t - LW)
        qs = q[t0:t0 + TR]
        kl = k[ls:ls + LW]
        vl = v[ls:ls + LW]

        row_ids = t0 + jax.lax.broadcasted_iota(jnp.int32, (TR, LW), 0)
        key_ids = ls + jax.lax.broadcasted_iota(jnp.int32, (TR, LW), 1)
        band = jnp.abs(key_ids - row_ids) <= WINDOW

        s_loc = jax.lax.dot_general(
            qs, kl, (((1,), (1,)), ((), ())),
            preferred_element_type=jnp.float32,
        ) * scale
        s_loc = jnp.where(band, s_loc, NEG)

        s_g = jax.lax.dot_general(
            qs, kg, (((1,), (1,)), ((), ())),
            preferred_element_type=jnp.float32,
        ) * scale
        s_g = jnp.where(gv, s_g, NEG)

        m = jnp.maximum(
            jnp.max(s_loc, axis=1, keepdims=True),
            jnp.max(s_g, axis=1, keepdims=True),
        )
        p_loc = jnp.exp(s_loc - m)
        p_g = jnp.exp(s_g - m)
        l = (jnp.sum(p_loc, axis=1, keepdims=True)
             + jnp.sum(p_g, axis=1, keepdims=True))
        acc = (
            jnp.dot(p_loc.astype(jnp.bfloat16), vl,
                    preferred_element_type=jnp.float32)
            + jnp.dot(p_g.astype(jnp.bfloat16), vg,
                      preferred_element_type=jnp.float32)
        )
        ctx_rows.append(acc / l)

    ctx = jnp.concatenate(ctx_rows, axis=0)
    contrib = jnp.dot(ctx.astype(jnp.bfloat16), wo_ref[...],
                      preferred_element_type=jnp.float32)

    @pl.when(h == 0)
    def _init():
        o_ref[...] = contrib + bo_ref[...]

    @pl.when(h != 0)
    def _accum():
        o_ref[...] += contrib


def _attention(qkv, kvsrc, gvalid, wo, bo):
    t = qkv.shape[0]
    d = H * HD
    ng = gvalid.shape[1]
    qkv4 = qkv.reshape(t, 3 * H, 1, HD)
    kvsrc4 = kvsrc.reshape(kvsrc.shape[0], 3 * H, 1, HD)
    grid = (H,)
    return pl.pallas_call(
        functools.partial(_attn_kernel, t=t),
        grid=grid,
        in_specs=[
            pl.BlockSpec((t, 1, 1, HD), lambda h: (0, h, 0, 0)),
            pl.BlockSpec((t, 1, 1, HD), lambda h: (0, H + h, 0, 0)),
            pl.BlockSpec((t, 1, 1, HD), lambda h: (0, 2 * H + h, 0, 0)),
            pl.BlockSpec((ng, 1, 1, HD), lambda h: (0, H + h, 0, 0)),
            pl.BlockSpec((ng, 1, 1, HD), lambda h: (0, 2 * H + h, 0, 0)),
            pl.BlockSpec((1, ng), lambda h: (0, 0)),
            pl.BlockSpec((HD, d), lambda h: (h, 0)),
            pl.BlockSpec((1, d), lambda h: (0, 0)),
        ],
        out_specs=pl.BlockSpec((t, d), lambda h: (0, 0)),
        out_shape=jax.ShapeDtypeStruct((t, d), jnp.float32),
        interpret=_INTERPRET,
    )(qkv4, qkv4, qkv4, kvsrc4, kvsrc4, gvalid, wo, bo)


def kernel(x, global_mask, Wq, bq, Wk, bk, Wv, bv, Wo, bo):
    b, t, d = x.shape
    x2 = x[0].astype(jnp.bfloat16)
    wqkv = jnp.concatenate([Wq, Wk, Wv], axis=1).astype(jnp.bfloat16)
    bqkv = jnp.concatenate([bq, bk, bv])[None, :]
    qkv = _matmul(x2, wqkv, bqkv, out_dtype=jnp.bfloat16)

    mask = global_mask[0]
    csum = jnp.cumsum(mask.astype(jnp.int32))
    g = csum[-1]
    slots = jnp.where(mask, csum - 1, GMAX + t)
    gidx = (
        jnp.zeros((GMAX,), jnp.int32)
        .at[slots]
        .set(jnp.arange(t, dtype=jnp.int32), mode="drop")
    )
    gvalid_fast = (jnp.arange(GMAX) < g).astype(jnp.float32)[None, :]
    gvalid_slow = mask.astype(jnp.float32)[None, :]

    wo_b = Wo.astype(jnp.bfloat16)
    bo_b = bo[None, :]

    def fast(qkv_):
        kv_glob = _gather_rows(qkv_, gidx)
        return _attention(qkv_, kv_glob, gvalid_fast, wo_b, bo_b)

    def slow(qkv_):
        return _attention(qkv_, qkv_, gvalid_slow, wo_b, bo_b)

    out2 = jax.lax.cond(g <= GMAX, fast, slow, qkv)
    return out2[None]


# speedup vs baseline: 1.4677x; 1.4677x over previous
import functools

import jax
import jax.numpy as jnp
import numpy as np
from jax.experimental import pallas as pl
from jax.experimental.pallas import tpu as pltpu
from jax.experimental.pallas import tpu_sc as plsc

H = 16
HD = 64
WINDOW = 8
NEG = -1e30
TR = 256
LW = 384
GMAX = 128

_INTERPRET = False


def _qkv_kernel(x_ref, w_ref, b_ref, gidx_ref, o_ref, og_ref):
    acc = (
        jnp.dot(x_ref[...], w_ref[...], preferred_element_type=jnp.float32)
        + b_ref[...]
    )
    out = acc.astype(o_ref.dtype)
    o_ref[...] = out
    m = x_ref.shape[0]
    g = gidx_ref.shape[1]
    gcol = jnp.transpose(gidx_ref[...])
    p = (jax.lax.broadcasted_iota(jnp.int32, (g, m), 1) == gcol)
    og_ref[...] = jnp.dot(
        p.astype(jnp.bfloat16), out, preferred_element_type=jnp.float32
    ).astype(og_ref.dtype)


def _qkv_matmul(x, w, b, gidx2, bn=1024):
    m, k = x.shape
    k2, n = w.shape
    g = gidx2.shape[1]
    grid = (n // bn,)
    return pl.pallas_call(
        _qkv_kernel,
        grid=grid,
        in_specs=[
            pl.BlockSpec((m, k), lambda j: (0, 0)),
            pl.BlockSpec((k, bn), lambda j: (0, j)),
            pl.BlockSpec((1, bn), lambda j: (0, j)),
            pl.BlockSpec((1, g), lambda j: (0, 0)),
        ],
        out_specs=[
            pl.BlockSpec((m, bn), lambda j: (0, j)),
            pl.BlockSpec((g, bn), lambda j: (0, j)),
        ],
        out_shape=[
            jax.ShapeDtypeStruct((m, n), jnp.bfloat16),
            jax.ShapeDtypeStruct((g, n), jnp.bfloat16),
        ],
        interpret=_INTERPRET,
    )(x, w, b, gidx2)


def _matmul_kernel(x_ref, w_ref, b_ref, o_ref):
    acc = (
        jnp.dot(x_ref[...], w_ref[...], preferred_element_type=jnp.float32)
        + b_ref[...]
    )
    o_ref[...] = acc.astype(o_ref.dtype)


def _attn_kernel(q_ref, k_ref, v_ref, kg_ref, vg_ref, gv_ref, wo_ref, bo_ref,
                 o_ref, *, t):
    scale = 1.0 / np.sqrt(HD)
    h = pl.program_id(0)
    nr = t // TR

    q = q_ref[...].reshape(t, HD)
    k = k_ref[...].reshape(t, HD)
    v = v_ref[...].reshape(t, HD)
    kg = kg_ref[...].reshape(kg_ref.shape[0], HD)
    vg = vg_ref[...].reshape(vg_ref.shape[0], HD)
    gv = gv_ref[...] > 0.0

    ctx_rows = []
    for r in range(nr):
        t0 = r * TR
        ls = min(max(t0 - (LW - TR) // 2, 0), t - LW)
        qs = q[t0:t0 + TR]
        kl = k[ls:ls + LW]
        vl = v[ls:ls + LW]

        row_ids = t0 + jax.lax.broadcasted_iota(jnp.int32, (TR, LW), 0)
        key_ids = ls + jax.lax.broadcasted_iota(jnp.int32, (TR, LW), 1)
        band = jnp.abs(key_ids - row_ids) <= WINDOW

        s_loc = jax.lax.dot_general(
            qs, kl, (((1,), (1,)), ((), ())),
            preferred_element_type=jnp.float32,
        ) * scale
        s_loc = jnp.where(band, s_loc, NEG)

        s_g = jax.lax.dot_general(
            qs, kg, (((1,), (1,)), ((), ())),
            preferred_element_type=jnp.float32,
        ) * scale
        s_g = jnp.where(gv, s_g, NEG)

        m = jnp.maximum(
            jnp.max(s_loc, axis=1, keepdims=True),
            jnp.max(s_g, axis=1, keepdims=True),
        )
        p_loc = jnp.exp(s_loc - m)
        p_g = jnp.exp(s_g - m)
        l = (jnp.sum(p_loc, axis=1, keepdims=True)
             + jnp.sum(p_g, axis=1, keepdims=True))
        acc = (
            jnp.dot(p_loc.astype(jnp.bfloat16), vl,
                    preferred_element_type=jnp.float32)
            + jnp.dot(p_g.astype(jnp.bfloat16), vg,
                      preferred_element_type=jnp.float32)
        )
        ctx_rows.append(acc / l)

    ctx = jnp.concatenate(ctx_rows, axis=0)
    contrib = jnp.dot(ctx.astype(jnp.bfloat16), wo_ref[...],
                      preferred_element_type=jnp.float32)

    @pl.when(h == 0)
    def _init():
        o_ref[...] = contrib + bo_ref[...]

    @pl.when(h != 0)
    def _accum():
        o_ref[...] += contrib


def _attention(qkv, kvsrc, gvalid, wo, bo):
    t = qkv.shape[0]
    d = H * HD
    ng = gvalid.shape[1]
    qkv4 = qkv.reshape(t, 3 * H, 1, HD)
    kvsrc4 = kvsrc.reshape(kvsrc.shape[0], 3 * H, 1, HD)
    grid = (H,)
    return pl.pallas_call(
        functools.partial(_attn_kernel, t=t),
        grid=grid,
        in_specs=[
            pl.BlockSpec((t, 1, 1, HD), lambda h: (0, h, 0, 0)),
            pl.BlockSpec((t, 1, 1, HD), lambda h: (0, H + h, 0, 0)),
            pl.BlockSpec((t, 1, 1, HD), lambda h: (0, 2 * H + h, 0, 0)),
            pl.BlockSpec((ng, 1, 1, HD), lambda h: (0, H + h, 0, 0)),
            pl.BlockSpec((ng, 1, 1, HD), lambda h: (0, 2 * H + h, 0, 0)),
            pl.BlockSpec((1, ng), lambda h: (0, 0)),
            pl.BlockSpec((HD, d), lambda h: (h, 0)),
            pl.BlockSpec((1, d), lambda h: (0, 0)),
        ],
        out_specs=pl.BlockSpec((t, d), lambda h: (0, 0)),
        out_shape=jax.ShapeDtypeStruct((t, d), jnp.float32),
        interpret=_INTERPRET,
    )(qkv4, qkv4, qkv4, kvsrc4, kvsrc4, gvalid, wo, bo)


def kernel(x, global_mask, Wq, bq, Wk, bk, Wv, bv, Wo, bo):
    b, t, d = x.shape
    x2 = x[0].astype(jnp.bfloat16)
    wqkv = jnp.concatenate([Wq, Wk, Wv], axis=1).astype(jnp.bfloat16)
    bqkv = jnp.concatenate([bq, bk, bv])[None, :]

    mask = global_mask[0]
    csum = jnp.cumsum(mask.astype(jnp.int32))
    g = csum[-1]
    slots = jnp.where(mask, csum - 1, GMAX + t)
    gidx = (
        jnp.zeros((GMAX,), jnp.int32)
        .at[slots]
        .set(jnp.arange(t, dtype=jnp.int32), mode="drop")
    )
    gvalid_fast = (jnp.arange(GMAX) < g).astype(jnp.float32)[None, :]
    gvalid_slow = mask.astype(jnp.float32)[None, :]

    qkv, kv_glob = _qkv_matmul(x2, wqkv, bqkv, gidx[None, :])

    wo_b = Wo.astype(jnp.bfloat16)
    bo_b = bo[None, :]

    def fast(qkv_):
        return _attention(qkv_, kv_glob, gvalid_fast, wo_b, bo_b)

    def slow(qkv_):
        return _attention(qkv_, qkv_, gvalid_slow, wo_b, bo_b)

    out2 = jax.lax.cond(g <= GMAX, fast, slow, qkv)
    return out2[None]
